# baseline (speedup 1.0000x reference)
import functools

import jax
import jax.numpy as jnp
from jax import lax
from jax.experimental import pallas as pl
from jax.experimental.pallas import tpu as pltpu
from jax.experimental.pallas import tpu_sc as plsc


def _time_mix_weights(T, dtype):
    tg = jnp.linspace(-1.0, 1.0, T, dtype=dtype)
    iy = (tg + 1.0) * 0.5 * (T - 1)
    iy0 = jnp.floor(iy)
    wy1 = iy - iy0
    wy0 = 1.0 - wy1
    v0 = (iy0 >= 0) & (iy0 <= T - 1)
    v1 = (iy0 + 1.0 >= 0) & (iy0 + 1.0 <= T - 1)
    wy0 = jnp.where(v0, wy0, jnp.zeros_like(wy0))
    wy1 = jnp.where(v1, wy1, jnp.zeros_like(wy1))
    s = iy0.astype(jnp.int32) == jnp.arange(T, dtype=jnp.int32)
    zero = jnp.zeros_like(wy0)
    wA = jnp.where(s, zero, wy0)
    wB = jnp.where(s, wy0, wy1)
    wC = jnp.where(s, wy1, zero)
    return wA, wB, wC


def _freq_warp_matrix(alpha, D, dtype):
    f = jnp.linspace(0.0, 1.0, D, dtype=dtype)
    warped = f ** alpha
    gx = warped * 2.0 - 1.0
    ix = (gx + 1.0) * 0.5 * (D - 1)
    ix0 = jnp.floor(ix)
    wx1 = ix - ix0
    wx0 = 1.0 - wx1
    v0 = (ix0 >= 0) & (ix0 <= D - 1)
    v1 = (ix0 + 1.0 >= 0) & (ix0 + 1.0 <= D - 1)
    wx0 = jnp.where(v0, wx0, jnp.zeros_like(wx0))
    wx1 = jnp.where(v1, wx1, jnp.zeros_like(wx1))
    i0 = jnp.clip(ix0, 0, D - 1).astype(jnp.int32)
    i1 = jnp.clip(ix0 + 1.0, 0, D - 1).astype(jnp.int32)
    k = jnp.arange(D, dtype=jnp.int32)[:, None]
    M = (wx0[None, :] * (k == i0[None, :]).astype(dtype)
         + wx1[None, :] * (k == i1[None, :]).astype(dtype))
    return M


def _tc_body(x_ref, w_ref, m_ref, o_ref, prev_ref, plast_ref):
    i = pl.program_id(0)

    @pl.when(i > 0)
    def _compute():
        prev = prev_ref[...]
        xm = jnp.concatenate([plast_ref[...], prev[:-1, :]], axis=0)
        xp = jnp.concatenate([prev[1:, :], x_ref[0:1, :]], axis=0)
        wA = w_ref[:, 0:1]
        wB = w_ref[:, 1:2]
        wC = w_ref[:, 2:3]
        mixed = wA * xm + wB * prev + wC * xp
        o_ref[...] = jax.lax.dot_general(
            mixed, m_ref[...], (((1,), (0,)), ((), ())),
            precision=jax.lax.Precision.HIGHEST,
            preferred_element_type=jnp.float32)
        plast_ref[...] = prev[-1:, :]

    @pl.when(i == 0)
    def _init():
        plast_ref[...] = jnp.zeros_like(plast_ref)

    prev_ref[...] = x_ref[...]


def _tc_warp(x2, w, M, block_t):
    T, D = x2.shape
    nb = T // block_t
    return pl.pallas_call(
        _tc_body,
        grid=(nb + 1,),
        in_specs=[
            pl.BlockSpec((block_t, D), lambda i: (jnp.minimum(i, nb - 1), 0)),
            pl.BlockSpec((block_t, 3), lambda i: (jnp.maximum(i - 1, 0), 0)),
            pl.BlockSpec((D, D), lambda i: (0, 0)),
        ],
        out_specs=pl.BlockSpec((block_t, D), lambda i: (jnp.maximum(i - 1, 0), 0)),
        out_shape=jax.ShapeDtypeStruct((T, D), x2.dtype),
        scratch_shapes=[
            pltpu.VMEM((block_t, D), x2.dtype),
            pltpu.VMEM((1, D), x2.dtype),
        ],
    )(x2, w, M)


def _freq_gather_tables(alpha, D, dtype):
    f = jnp.linspace(0.0, 1.0, D, dtype=dtype)
    warped = f ** alpha
    gx = warped * 2.0 - 1.0
    ix = (gx + 1.0) * 0.5 * (D - 1)
    ix0 = jnp.floor(ix)
    wx1 = ix - ix0
    wx0 = 1.0 - wx1
    v0 = (ix0 >= 0) & (ix0 <= D - 1)
    v1 = (ix0 + 1.0 >= 0) & (ix0 + 1.0 <= D - 1)
    wx0 = jnp.where(v0, wx0, jnp.zeros_like(wx0))
    wx1 = jnp.where(v1, wx1, jnp.zeros_like(wx1))
    j0 = jnp.clip(ix0, 0, D - 1).astype(jnp.int32)
    j1 = jnp.clip(ix0 + 1.0, 0, D - 1).astype(jnp.int32)
    nv = D // 16
    jidx = jnp.concatenate([j0, j1]).reshape(2 * nv, 16)
    wx = jnp.concatenate([wx0, wx1]).reshape(2 * nv, 16).astype(dtype)
    return jidx, wx


_SC_NW = 32
_SC_W = 128
_SC_D = 80


def _sc_body(x_hbm, wa_hbm, wb_hbm, wc_hbm, j0_hbm, j1_hbm, wx0_hbm, wx1_hbm,
             o_hbm,
             xb0, xb1, wab0, wab1, wbb0, wbb1, wcb0, wcb1,
             ob0, ob1, j0buf, j1buf, wx0buf, wx1buf,
             sin0, sin1, sout0, sout1):
    D = _SC_D
    T = x_hbm.shape[1]
    W = _SC_W
    WH = 3 * W
    nch = T // W
    n_iter = (nch + _SC_NW - 1) // _SC_NW
    wid = lax.axis_index("s") * 2 + lax.axis_index("c")

    pltpu.sync_copy(j0_hbm, j0buf)
    pltpu.sync_copy(j1_hbm, j1buf)
    pltpu.sync_copy(wx0_hbm, wx0buf)
    pltpu.sync_copy(wx1_hbm, wx1buf)

    xbufs, obufs = [xb0, xb1], [ob0, ob1]
    wabufs, wbbufs, wcbufs = [wab0, wab1], [wbb0, wbb1], [wcb0, wcb1]
    sins, souts = [sin0, sin1], [sout0, sout1]

    def start_in(i, p):
        ch = wid + i * _SC_NW
        c0 = ch * W
        xb, sem = xbufs[p], sins[p]
        if i == 0:
            @pl.when(ch == 0)
            def _():
                pltpu.async_copy(x_hbm.at[:, pl.ds(0, 2 * W)],
                                 xb.at[:, pl.ds(W, 2 * W)], sem)
                pltpu.async_copy(x_hbm.at[:, pl.ds(0, W)],
                                 xb.at[:, pl.ds(0, W)], sem)

            @pl.when(ch > 0)
            def _():
                pltpu.async_copy(x_hbm.at[:, pl.ds(c0 - W, WH)], xb, sem)
        elif i == n_iter - 1:
            @pl.when(ch == nch - 1)
            def _():
                pltpu.async_copy(x_hbm.at[:, pl.ds(T - 2 * W, 2 * W)],
                                 xb.at[:, pl.ds(0, 2 * W)], sem)
                pltpu.async_copy(x_hbm.at[:, pl.ds(T - W, W)],
                                 xb.at[:, pl.ds(2 * W, W)], sem)

            @pl.when(ch < nch - 1)
            def _():
                pltpu.async_copy(x_hbm.at[:, pl.ds(c0 - W, WH)], xb, sem)
        else:
            pltpu.async_copy(x_hbm.at[:, pl.ds(c0 - W, WH)], xb, sem)
        pltpu.async_copy(wa_hbm.at[pl.ds(c0, W)], wabufs[p], sem)
        pltpu.async_copy(wb_hbm.at[pl.ds(c0, W)], wbbufs[p], sem)
        pltpu.async_copy(wc_hbm.at[pl.ds(c0, W)], wcbufs[p], sem)

    def wait_in(p):
        pltpu.make_async_copy(x_hbm.at[:, pl.ds(0, WH)], xbufs[p],
                              sins[p]).wait()
        pltpu.make_async_copy(wa_hbm.at[pl.ds(0, W)], wabufs[p],
                              sins[p]).wait()
        pltpu.make_async_copy(wb_hbm.at[pl.ds(0, W)], wbbufs[p],
                              sins[p]).wait()
        pltpu.make_async_copy(wc_hbm.at[pl.ds(0, W)], wcbufs[p],
                              sins[p]).wait()

    def wait_out(p):
        pltpu.make_async_copy(obufs[p], o_hbm.at[:, pl.ds(0, W)],
                              souts[p]).wait()

    def compute(i, p):
        xb, ob = xbufs[p], obufs[p]
        wab, wbb, wcb = wabufs[p], wbbufs[p], wcbufs[p]

        @plsc.parallel_loop(0, W // 16, unroll=1)
        def _cols(g):
            wAv = wab[pl.ds(g * 16, 16)]
            wBv = wbb[pl.ds(g * 16, 16)]
            wCv = wcb[pl.ds(g * 16, 16)]
            c = W + g * 16
            lanes = jax.lax.iota(jnp.int32, 16)
            cm1 = lanes + (c - 1)
            cp1 = lanes + (c + 1)

            @pl.loop(0, D // 16)
            def _rows(dd):
                j0v = j0buf[pl.ds(dd * 16, 16)]
                j1v = j1buf[pl.ds(dd * 16, 16)]
                wx0v = wx0buf[pl.ds(dd * 16, 16)]
                wx1v = wx1buf[pl.ds(dd * 16, 16)]
                for l in range(16):
                    d = dd * 16 + l
                    j0d = j0v[l]
                    j1d = j1v[l]
                    r0 = jnp.full((16,), j0d, jnp.int32)
                    r1 = jnp.full((16,), j1d, jnp.int32)
                    m0 = (wAv * plsc.load_gather(xb, [r0, cm1])
                          + wBv * xb[j0d, pl.ds(c, 16)]
                          + wCv * plsc.load_gather(xb, [r0, cp1]))
                    m1 = (wAv * plsc.load_gather(xb, [r1, cm1])
                          + wBv * xb[j1d, pl.ds(c, 16)]
                          + wCv * plsc.load_gather(xb, [r1, cp1]))
                    ob[d, pl.ds(g * 16, 16)] = wx0v[l] * m0 + wx1v[l] * m1

        c0 = (wid + i * _SC_NW) * W
        pltpu.async_copy(ob, o_hbm.at[:, pl.ds(c0, W)], souts[p])

    def chunk_full(i):
        return i * _SC_NW + _SC_NW <= nch

    def guarded(i, fn):
        if chunk_full(i):
            fn()
        else:
            @pl.when(wid + i * _SC_NW < nch)
            def _():
                fn()

    start_in(0, 0)
    for i in range(n_iter):
        p = i & 1
        if i + 1 < n_iter:
            guarded(i + 1, functools.partial(start_in, i + 1, 1 - p))
        if i >= 2:
            guarded(i - 2, functools.partial(wait_out, p))
        def _work(i=i, p=p):
            wait_in(p)
            compute(i, p)
        guarded(i, _work)
    for i in (n_iter - 2, n_iter - 1):
        if i >= 0:
            guarded(i, functools.partial(wait_out, i & 1))


def _sc_warp(xT, wA, wB, wC, j0, j1, wx0, wx1):
    D, T = xT.shape
    mesh = plsc.VectorSubcoreMesh(core_axis_name="c", subcore_axis_name="s")
    k = pl.kernel(
        _sc_body,
        out_type=jax.ShapeDtypeStruct((D, T), xT.dtype),
        mesh=mesh,
        compiler_params=pltpu.CompilerParams(needs_layout_passes=False),
        scratch_types=[
            pltpu.VMEM((_SC_D, 3 * _SC_W), xT.dtype),
            pltpu.VMEM((_SC_D, 3 * _SC_W), xT.dtype),
            pltpu.VMEM((_SC_W,), xT.dtype),
            pltpu.VMEM((_SC_W,), xT.dtype),
            pltpu.VMEM((_SC_W,), xT.dtype),
            pltpu.VMEM((_SC_W,), xT.dtype),
            pltpu.VMEM((_SC_W,), xT.dtype),
            pltpu.VMEM((_SC_W,), xT.dtype),
            pltpu.VMEM((_SC_D, _SC_W), xT.dtype),
            pltpu.VMEM((_SC_D, _SC_W), xT.dtype),
            pltpu.VMEM((_SC_D,), jnp.int32),
            pltpu.VMEM((_SC_D,), jnp.int32),
            pltpu.VMEM((_SC_D,), xT.dtype),
            pltpu.VMEM((_SC_D,), xT.dtype),
            pltpu.SemaphoreType.DMA,
            pltpu.SemaphoreType.DMA,
            pltpu.SemaphoreType.DMA,
            pltpu.SemaphoreType.DMA,
        ],
    )
    return k(xT, wA, wB, wC, j0, j1, wx0, wx1)


def _freq_tap_tables(alpha, D, dtype):
    f = jnp.linspace(0.0, 1.0, D, dtype=dtype)
    warped = f ** alpha
    gx = warped * 2.0 - 1.0
    ix = (gx + 1.0) * 0.5 * (D - 1)
    ix0 = jnp.floor(ix)
    wx1 = ix - ix0
    wx0 = 1.0 - wx1
    v0 = (ix0 >= 0) & (ix0 <= D - 1)
    v1 = (ix0 + 1.0 >= 0) & (ix0 + 1.0 <= D - 1)
    wx0 = jnp.where(v0, wx0, jnp.zeros_like(wx0))
    wx1 = jnp.where(v1, wx1, jnp.zeros_like(wx1))
    j0 = jnp.clip(ix0, 0, D - 1).astype(jnp.int32)
    j1 = jnp.clip(ix0 + 1.0, 0, D - 1).astype(jnp.int32)
    return j0, j1, wx0.astype(dtype), wx1.astype(dtype)


def kernel(x, alpha1_raw):
    B, T, D = x.shape
    assert B == 1
    alpha = jnp.reshape(alpha1_raw, ())
    wA, wB, wC = _time_mix_weights(T, x.dtype)
    j0, j1, wx0, wx1 = _freq_tap_tables(alpha, D, x.dtype)
    xT = jnp.transpose(x.reshape(T, D))
    outT = _sc_warp(xT, wA, wB, wC, j0, j1, wx0, wx1)
    return jnp.transpose(outT).reshape(B, T, D)


# speedup vs baseline: 1.0247x; 1.0247x over previous
import functools

import jax
import jax.numpy as jnp
from jax import lax
from jax.experimental import pallas as pl
from jax.experimental.pallas import tpu as pltpu
from jax.experimental.pallas import tpu_sc as plsc


def _time_mix_weights(T, dtype):
    tg = jnp.linspace(-1.0, 1.0, T, dtype=dtype)
    iy = (tg + 1.0) * 0.5 * (T - 1)
    iy0 = jnp.floor(iy)
    wy1 = iy - iy0
    wy0 = 1.0 - wy1
    v0 = (iy0 >= 0) & (iy0 <= T - 1)
    v1 = (iy0 + 1.0 >= 0) & (iy0 + 1.0 <= T - 1)
    wy0 = jnp.where(v0, wy0, jnp.zeros_like(wy0))
    wy1 = jnp.where(v1, wy1, jnp.zeros_like(wy1))
    s = iy0.astype(jnp.int32) == jnp.arange(T, dtype=jnp.int32)
    zero = jnp.zeros_like(wy0)
    wA = jnp.where(s, zero, wy0)
    wB = jnp.where(s, wy0, wy1)
    wC = jnp.where(s, wy1, zero)
    return wA, wB, wC


def _freq_warp_matrix(alpha, D, dtype):
    f = jnp.linspace(0.0, 1.0, D, dtype=dtype)
    warped = f ** alpha
    gx = warped * 2.0 - 1.0
    ix = (gx + 1.0) * 0.5 * (D - 1)
    ix0 = jnp.floor(ix)
    wx1 = ix - ix0
    wx0 = 1.0 - wx1
    v0 = (ix0 >= 0) & (ix0 <= D - 1)
    v1 = (ix0 + 1.0 >= 0) & (ix0 + 1.0 <= D - 1)
    wx0 = jnp.where(v0, wx0, jnp.zeros_like(wx0))
    wx1 = jnp.where(v1, wx1, jnp.zeros_like(wx1))
    i0 = jnp.clip(ix0, 0, D - 1).astype(jnp.int32)
    i1 = jnp.clip(ix0 + 1.0, 0, D - 1).astype(jnp.int32)
    k = jnp.arange(D, dtype=jnp.int32)[:, None]
    M = (wx0[None, :] * (k == i0[None, :]).astype(dtype)
         + wx1[None, :] * (k == i1[None, :]).astype(dtype))
    return M


def _tc_body(x_ref, w_ref, m_ref, o_ref, prev_ref, plast_ref):
    i = pl.program_id(0)

    @pl.when(i > 0)
    def _compute():
        prev = prev_ref[...]
        xm = jnp.concatenate([plast_ref[...], prev[:-1, :]], axis=0)
        xp = jnp.concatenate([prev[1:, :], x_ref[0:1, :]], axis=0)
        wA = w_ref[:, 0:1]
        wB = w_ref[:, 1:2]
        wC = w_ref[:, 2:3]
        mixed = wA * xm + wB * prev + wC * xp
        o_ref[...] = jax.lax.dot_general(
            mixed, m_ref[...], (((1,), (0,)), ((), ())),
            precision=jax.lax.Precision.HIGHEST,
            preferred_element_type=jnp.float32)
        plast_ref[...] = prev[-1:, :]

    @pl.when(i == 0)
    def _init():
        plast_ref[...] = jnp.zeros_like(plast_ref)

    prev_ref[...] = x_ref[...]


def _tc_warp(x2, w, M, block_t):
    T, D = x2.shape
    nb = T // block_t
    return pl.pallas_call(
        _tc_body,
        grid=(nb + 1,),
        in_specs=[
            pl.BlockSpec((block_t, D), lambda i: (jnp.minimum(i, nb - 1), 0)),
            pl.BlockSpec((block_t, 3), lambda i: (jnp.maximum(i - 1, 0), 0)),
            pl.BlockSpec((D, D), lambda i: (0, 0)),
        ],
        out_specs=pl.BlockSpec((block_t, D), lambda i: (jnp.maximum(i - 1, 0), 0)),
        out_shape=jax.ShapeDtypeStruct((T, D), x2.dtype),
        scratch_shapes=[
            pltpu.VMEM((block_t, D), x2.dtype),
            pltpu.VMEM((1, D), x2.dtype),
        ],
    )(x2, w, M)


def _freq_gather_tables(alpha, D, dtype):
    f = jnp.linspace(0.0, 1.0, D, dtype=dtype)
    warped = f ** alpha
    gx = warped * 2.0 - 1.0
    ix = (gx + 1.0) * 0.5 * (D - 1)
    ix0 = jnp.floor(ix)
    wx1 = ix - ix0
    wx0 = 1.0 - wx1
    v0 = (ix0 >= 0) & (ix0 <= D - 1)
    v1 = (ix0 + 1.0 >= 0) & (ix0 + 1.0 <= D - 1)
    wx0 = jnp.where(v0, wx0, jnp.zeros_like(wx0))
    wx1 = jnp.where(v1, wx1, jnp.zeros_like(wx1))
    j0 = jnp.clip(ix0, 0, D - 1).astype(jnp.int32)
    j1 = jnp.clip(ix0 + 1.0, 0, D - 1).astype(jnp.int32)
    nv = D // 16
    jidx = jnp.concatenate([j0, j1]).reshape(2 * nv, 16)
    wx = jnp.concatenate([wx0, wx1]).reshape(2 * nv, 16).astype(dtype)
    return jidx, wx


_SC_NW = 32
_SC_W = 128
_SC_D = 80


def _sc_body(x_hbm, wa_hbm, wb_hbm, wc_hbm, j0_hbm, j1_hbm, wx0_hbm, wx1_hbm,
             o_hbm,
             xb0, xb1, wab0, wab1, wbb0, wbb1, wcb0, wcb1,
             ob0, ob1, j0buf, j1buf, wx0buf, wx1buf,
             sin0, sin1, sout0, sout1):
    D = _SC_D
    T = x_hbm.shape[1]
    W = _SC_W
    WH = 3 * W
    nch = T // W
    n_iter = (nch + _SC_NW - 1) // _SC_NW
    wid = lax.axis_index("s") * 2 + lax.axis_index("c")

    pltpu.sync_copy(j0_hbm, j0buf)
    pltpu.sync_copy(j1_hbm, j1buf)
    pltpu.sync_copy(wx0_hbm, wx0buf)
    pltpu.sync_copy(wx1_hbm, wx1buf)

    xbufs, obufs = [xb0, xb1], [ob0, ob1]
    wabufs, wbbufs, wcbufs = [wab0, wab1], [wbb0, wbb1], [wcb0, wcb1]
    sins, souts = [sin0, sin1], [sout0, sout1]

    def start_in(i, p):
        ch = wid + i * _SC_NW
        c0 = ch * W
        xb, sem = xbufs[p], sins[p]

        @pl.when(ch == 0)
        def _():
            pltpu.async_copy(x_hbm.at[:, pl.ds(0, 2 * W)],
                             xb.at[:, pl.ds(W, 2 * W)], sem)
            pltpu.async_copy(x_hbm.at[:, pl.ds(0, W)],
                             xb.at[:, pl.ds(0, W)], sem)

        @pl.when(ch == nch - 1)
        def _():
            pltpu.async_copy(x_hbm.at[:, pl.ds(T - 2 * W, 2 * W)],
                             xb.at[:, pl.ds(0, 2 * W)], sem)
            pltpu.async_copy(x_hbm.at[:, pl.ds(T - W, W)],
                             xb.at[:, pl.ds(2 * W, W)], sem)

        @pl.when((ch > 0) & (ch < nch - 1))
        def _():
            pltpu.async_copy(x_hbm.at[:, pl.ds(c0 - W, WH)], xb, sem)
        pltpu.async_copy(wa_hbm.at[pl.ds(c0, W)], wabufs[p], sem)
        pltpu.async_copy(wb_hbm.at[pl.ds(c0, W)], wbbufs[p], sem)
        pltpu.async_copy(wc_hbm.at[pl.ds(c0, W)], wcbufs[p], sem)

    def wait_in(p):
        pltpu.make_async_copy(x_hbm.at[:, pl.ds(0, WH)], xbufs[p],
                              sins[p]).wait()
        pltpu.make_async_copy(wa_hbm.at[pl.ds(0, W)], wabufs[p],
                              sins[p]).wait()
        pltpu.make_async_copy(wb_hbm.at[pl.ds(0, W)], wbbufs[p],
                              sins[p]).wait()
        pltpu.make_async_copy(wc_hbm.at[pl.ds(0, W)], wcbufs[p],
                              sins[p]).wait()

    def wait_out(p):
        pltpu.make_async_copy(obufs[p], o_hbm.at[:, pl.ds(0, W)],
                              souts[p]).wait()

    def compute(i, p):
        xb, ob = xbufs[p], obufs[p]
        wab, wbb, wcb = wabufs[p], wbbufs[p], wcbufs[p]

        @plsc.parallel_loop(0, W // 16, unroll=2)
        def _cols(g):
            wAv = wab[pl.ds(g * 16, 16)]
            wBv = wbb[pl.ds(g * 16, 16)]
            wCv = wcb[pl.ds(g * 16, 16)]
            c = W + g * 16
            lanes = jax.lax.iota(jnp.int32, 16)
            cm1 = lanes + (c - 1)
            cp1 = lanes + (c + 1)

            @pl.loop(0, D // 16)
            def _rows(dd):
                j0v = j0buf[pl.ds(dd * 16, 16)]
                j1v = j1buf[pl.ds(dd * 16, 16)]
                wx0v = wx0buf[pl.ds(dd * 16, 16)]
                wx1v = wx1buf[pl.ds(dd * 16, 16)]
                for l in range(16):
                    d = dd * 16 + l
                    j0d = j0v[l]
                    j1d = j1v[l]
                    r0 = jnp.full((16,), j0d, jnp.int32)
                    r1 = jnp.full((16,), j1d, jnp.int32)
                    m0 = (wAv * plsc.load_gather(xb, [r0, cm1])
                          + wBv * xb[j0d, pl.ds(c, 16)]
                          + wCv * plsc.load_gather(xb, [r0, cp1]))
                    m1 = (wAv * plsc.load_gather(xb, [r1, cm1])
                          + wBv * xb[j1d, pl.ds(c, 16)]
                          + wCv * plsc.load_gather(xb, [r1, cp1]))
                    ob[d, pl.ds(g * 16, 16)] = wx0v[l] * m0 + wx1v[l] * m1

        c0 = (wid + i * _SC_NW) * W
        pltpu.async_copy(ob, o_hbm.at[:, pl.ds(c0, W)], souts[p])

    def chunk_valid(i):
        return wid + i * _SC_NW < nch

    start_in(0, 0)
    start_in(1, 1)
    wait_in(0)
    compute(0, 0)

    @pl.loop(0, (n_iter - 2) // 2)
    def _pairs(k):
        i1 = 2 * k + 1
        start_in(i1 + 1, 0)
        @pl.when(i1 >= 2)
        def _():
            wait_out(1)
        wait_in(1)
        compute(i1, 1)

        i2 = 2 * k + 2

        @pl.when(chunk_valid(i2 + 1))
        def _():
            start_in(i2 + 1, 1)
        wait_out(0)
        wait_in(0)
        compute(i2, 0)

    wait_out(1)

    @pl.when(chunk_valid(n_iter - 1))
    def _():
        wait_in(1)
        compute(n_iter - 1, 1)

    wait_out(0)

    @pl.when(chunk_valid(n_iter - 1))
    def _():
        wait_out(1)


def _sc_warp(xT, wA, wB, wC, j0, j1, wx0, wx1):
    D, T = xT.shape
    mesh = plsc.VectorSubcoreMesh(core_axis_name="c", subcore_axis_name="s")
    k = pl.kernel(
        _sc_body,
        out_type=jax.ShapeDtypeStruct((D, T), xT.dtype),
        mesh=mesh,
        compiler_params=pltpu.CompilerParams(needs_layout_passes=False),
        scratch_types=[
            pltpu.VMEM((_SC_D, 3 * _SC_W), xT.dtype),
            pltpu.VMEM((_SC_D, 3 * _SC_W), xT.dtype),
            pltpu.VMEM((_SC_W,), xT.dtype),
            pltpu.VMEM((_SC_W,), xT.dtype),
            pltpu.VMEM((_SC_W,), xT.dtype),
            pltpu.VMEM((_SC_W,), xT.dtype),
            pltpu.VMEM((_SC_W,), xT.dtype),
            pltpu.VMEM((_SC_W,), xT.dtype),
            pltpu.VMEM((_SC_D, _SC_W), xT.dtype),
            pltpu.VMEM((_SC_D, _SC_W), xT.dtype),
            pltpu.VMEM((_SC_D,), jnp.int32),
            pltpu.VMEM((_SC_D,), jnp.int32),
            pltpu.VMEM((_SC_D,), xT.dtype),
            pltpu.VMEM((_SC_D,), xT.dtype),
            pltpu.SemaphoreType.DMA,
            pltpu.SemaphoreType.DMA,
            pltpu.SemaphoreType.DMA,
            pltpu.SemaphoreType.DMA,
        ],
    )
    return k(xT, wA, wB, wC, j0, j1, wx0, wx1)


def _freq_tap_tables(alpha, D, dtype):
    f = jnp.linspace(0.0, 1.0, D, dtype=dtype)
    warped = f ** alpha
    gx = warped * 2.0 - 1.0
    ix = (gx + 1.0) * 0.5 * (D - 1)
    ix0 = jnp.floor(ix)
    wx1 = ix - ix0
    wx0 = 1.0 - wx1
    v0 = (ix0 >= 0) & (ix0 <= D - 1)
    v1 = (ix0 + 1.0 >= 0) & (ix0 + 1.0 <= D - 1)
    wx0 = jnp.where(v0, wx0, jnp.zeros_like(wx0))
    wx1 = jnp.where(v1, wx1, jnp.zeros_like(wx1))
    j0 = jnp.clip(ix0, 0, D - 1).astype(jnp.int32)
    j1 = jnp.clip(ix0 + 1.0, 0, D - 1).astype(jnp.int32)
    return j0, j1, wx0.astype(dtype), wx1.astype(dtype)


def kernel(x, alpha1_raw):
    B, T, D = x.shape
    assert B == 1
    alpha = jnp.reshape(alpha1_raw, ())
    wA, wB, wC = _time_mix_weights(T, x.dtype)
    j0, j1, wx0, wx1 = _freq_tap_tables(alpha, D, x.dtype)
    xT = jnp.transpose(x.reshape(T, D))
    outT = _sc_warp(xT, wA, wB, wC, j0, j1, wx0, wx1)
    return jnp.transpose(outT).reshape(B, T, D)


# speedup vs baseline: 1.3578x; 1.3250x over previous
import functools

import jax
import jax.numpy as jnp
from jax import lax
from jax.experimental import pallas as pl
from jax.experimental.pallas import tpu as pltpu
from jax.experimental.pallas import tpu_sc as plsc


def _time_mix_weights(T, dtype):
    tg = jnp.linspace(-1.0, 1.0, T, dtype=dtype)
    iy = (tg + 1.0) * 0.5 * (T - 1)
    iy0 = jnp.floor(iy)
    wy1 = iy - iy0
    wy0 = 1.0 - wy1
    v0 = (iy0 >= 0) & (iy0 <= T - 1)
    v1 = (iy0 + 1.0 >= 0) & (iy0 + 1.0 <= T - 1)
    wy0 = jnp.where(v0, wy0, jnp.zeros_like(wy0))
    wy1 = jnp.where(v1, wy1, jnp.zeros_like(wy1))
    s = iy0.astype(jnp.int32) == jnp.arange(T, dtype=jnp.int32)
    zero = jnp.zeros_like(wy0)
    wA = jnp.where(s, zero, wy0)
    wB = jnp.where(s, wy0, wy1)
    wC = jnp.where(s, wy1, zero)
    return wA, wB, wC


def _freq_gather_tables(alpha, D, dtype):
    f = jnp.linspace(0.0, 1.0, D, dtype=dtype)
    warped = f ** alpha
    gx = warped * 2.0 - 1.0
    ix = (gx + 1.0) * 0.5 * (D - 1)
    ix0 = jnp.floor(ix)
    wx1 = ix - ix0
    wx0 = 1.0 - wx1
    v0 = (ix0 >= 0) & (ix0 <= D - 1)
    v1 = (ix0 + 1.0 >= 0) & (ix0 + 1.0 <= D - 1)
    wx0 = jnp.where(v0, wx0, jnp.zeros_like(wx0))
    wx1 = jnp.where(v1, wx1, jnp.zeros_like(wx1))
    j0 = jnp.clip(ix0, 0, D - 1).astype(jnp.int32)
    j1 = jnp.clip(ix0 + 1.0, 0, D - 1).astype(jnp.int32)
    nv = D // 16
    jidx = jnp.concatenate([j0, j1]).reshape(2 * nv, 16)
    wx = jnp.concatenate([wx0, wx1]).reshape(2 * nv, 16).astype(dtype)
    return jidx, wx


_SC_NW = 32
_SC_C = 128
_SC_D = 80
_SC_NV = 5


def _sc_body(x3_hbm, wa_hbm, wb_hbm, wc_hbm, j_hbm, wx_hbm, o3_hbm,
             xb0, xb1, wab0, wab1, wbb0, wbb1, wcb0, wcb1,
             ob0, ob1, warpbuf, jbuf, wxbuf,
             sin0, sin1, sout0, sout1):
    D = _SC_D
    T = x3_hbm.shape[1]
    x_hbm = x3_hbm.at[0]
    o_hbm = o3_hbm.at[0]
    C = _SC_C
    nch = T // C
    n_iter = (nch + _SC_NW - 1) // _SC_NW
    wid = lax.axis_index("s") * 2 + lax.axis_index("c")

    pltpu.sync_copy(j_hbm, jbuf)
    pltpu.sync_copy(wx_hbm, wxbuf)
    j0s = [jbuf[k] for k in range(_SC_NV)]
    j1s = [jbuf[_SC_NV + k] for k in range(_SC_NV)]
    wx0s = [wxbuf[k] for k in range(_SC_NV)]
    wx1s = [wxbuf[_SC_NV + k] for k in range(_SC_NV)]

    xbufs, obufs = [xb0, xb1], [ob0, ob1]
    wabufs, wbbufs, wcbufs = [wab0, wab1], [wbb0, wbb1], [wcb0, wcb1]
    sins, souts = [sin0, sin1], [sout0, sout1]

    def start_in(i, p):
        ch = wid + i * _SC_NW
        t0 = ch * C
        xb, sem = xbufs[p], sins[p]
        if i == 0:
            @pl.when(ch == 0)
            def _():
                pltpu.async_copy(x_hbm.at[pl.ds(0, C + 8)],
                                 xb.at[pl.ds(8, C + 8)], sem)
                pltpu.async_copy(x_hbm.at[pl.ds(0, 8)], xb.at[pl.ds(0, 8)],
                                 sem)

            @pl.when(ch > 0)
            def _():
                pltpu.async_copy(x_hbm.at[pl.ds(t0 - 8, C + 16)], xb, sem)
        elif i == n_iter - 1:
            @pl.when(ch == nch - 1)
            def _():
                pltpu.async_copy(x_hbm.at[pl.ds(T - C - 16, C + 16)], xb,
                                 sem)

            @pl.when(ch < nch - 1)
            def _():
                pltpu.async_copy(x_hbm.at[pl.ds(t0 - 8, C + 16)], xb, sem)
        else:
            pltpu.async_copy(x_hbm.at[pl.ds(t0 - 8, C + 16)], xb, sem)
        pltpu.async_copy(wa_hbm.at[pl.ds(t0, C)], wabufs[p], sem)
        pltpu.async_copy(wb_hbm.at[pl.ds(t0, C)], wbbufs[p], sem)
        pltpu.async_copy(wc_hbm.at[pl.ds(t0, C)], wcbufs[p], sem)

    def wait_in(p):
        pltpu.make_async_copy(x_hbm.at[pl.ds(0, C + 16)], xbufs[p],
                              sins[p]).wait()
        pltpu.make_async_copy(wa_hbm.at[pl.ds(0, C)], wabufs[p],
                              sins[p]).wait()
        pltpu.make_async_copy(wb_hbm.at[pl.ds(0, C)], wbbufs[p],
                              sins[p]).wait()
        pltpu.make_async_copy(wc_hbm.at[pl.ds(0, C)], wcbufs[p],
                              sins[p]).wait()

    def wait_out(p):
        pltpu.make_async_copy(obufs[p], o_hbm.at[pl.ds(0, C)],
                              souts[p]).wait()

    def compute(i, p):
        xb = xbufs[p]
        if i == n_iter - 1:
            ch = wid + i * _SC_NW
            bias = jnp.where(ch == nch - 1, 16, 8)
        else:
            bias = 8

        @plsc.parallel_loop(0, C + 4, unroll=4)
        def _warp(r):
            row = jnp.minimum(bias + r - 1, C + 15)
            rs = jnp.full((16,), row, jnp.int32)
            for k in range(_SC_NV):
                g0 = plsc.load_gather(xb, [rs, j0s[k]])
                g1 = plsc.load_gather(xb, [rs, j1s[k]])
                warpbuf[r, pl.ds(k * 16, 16)] = wx0s[k] * g0 + wx1s[k] * g1

        wab, wbb, wcb, ob = wabufs[p], wbbufs[p], wcbufs[p], obufs[p]

        @plsc.parallel_loop(0, C, unroll=4)
        def _mix(r):
            rs = jnp.full((16,), r, jnp.int32)
            wA = plsc.load_gather(wab, [rs])
            wB = plsc.load_gather(wbb, [rs])
            wC = plsc.load_gather(wcb, [rs])
            for k in range(_SC_NV):
                sl = pl.ds(k * 16, 16)
                ob[r, sl] = (wA * warpbuf[r, sl] + wB * warpbuf[r + 1, sl]
                             + wC * warpbuf[r + 2, sl])

        t0 = (wid + i * _SC_NW) * C
        pltpu.async_copy(ob, o_hbm.at[pl.ds(t0, C)], souts[p])

    def chunk_full(i):
        return i * _SC_NW + _SC_NW <= nch

    def guarded(i, fn):
        if chunk_full(i):
            fn()
        else:
            @pl.when(wid + i * _SC_NW < nch)
            def _():
                fn()

    start_in(0, 0)
    for i in range(n_iter):
        p = i & 1
        if i + 1 < n_iter:
            guarded(i + 1, functools.partial(start_in, i + 1, 1 - p))
        if i >= 2:
            guarded(i - 2, functools.partial(wait_out, p))
        def _work(i=i, p=p):
            wait_in(p)
            compute(i, p)
        guarded(i, _work)
    for i in (n_iter - 2, n_iter - 1):
        if i >= 0:
            guarded(i, functools.partial(wait_out, i & 1))


def _sc_warp(x3, wA, wB, wC, jidx, wx):
    _, T, D = x3.shape
    x2 = x3
    mesh = plsc.VectorSubcoreMesh(core_axis_name="c", subcore_axis_name="s")
    k = pl.kernel(
        _sc_body,
        out_type=jax.ShapeDtypeStruct((1, T, D), x2.dtype),
        mesh=mesh,
        compiler_params=pltpu.CompilerParams(needs_layout_passes=False),
        scratch_types=[
            pltpu.VMEM((_SC_C + 16, _SC_D), x2.dtype),
            pltpu.VMEM((_SC_C + 16, _SC_D), x2.dtype),
            pltpu.VMEM((_SC_C,), x2.dtype),
            pltpu.VMEM((_SC_C,), x2.dtype),
            pltpu.VMEM((_SC_C,), x2.dtype),
            pltpu.VMEM((_SC_C,), x2.dtype),
            pltpu.VMEM((_SC_C,), x2.dtype),
            pltpu.VMEM((_SC_C,), x2.dtype),
            pltpu.VMEM((_SC_C, _SC_D), x2.dtype),
            pltpu.VMEM((_SC_C, _SC_D), x2.dtype),
            pltpu.VMEM((_SC_C + 4, _SC_D), x2.dtype),
            pltpu.VMEM((2 * _SC_NV, 16), jnp.int32),
            pltpu.VMEM((2 * _SC_NV, 16), x2.dtype),
            pltpu.SemaphoreType.DMA,
            pltpu.SemaphoreType.DMA,
            pltpu.SemaphoreType.DMA,
            pltpu.SemaphoreType.DMA,
        ],
    )
    return k(x3, wA, wB, wC, jidx, wx)


def kernel(x, alpha1_raw):
    B, T, D = x.shape
    assert B == 1
    alpha = jnp.reshape(alpha1_raw, ())
    wA, wB, wC = _time_mix_weights(T, x.dtype)
    jidx, wx = _freq_gather_tables(alpha, D, x.dtype)
    return _sc_warp(x, wA, wB, wC, jidx, wx)
